# baseline (speedup 1.0000x reference)
import functools

import jax
import jax.numpy as jnp
import numpy as np
from jax.experimental import pallas as pl
from jax.experimental.pallas import tpu as pltpu

N_EXPERTS = 64
N_GROUPS = 4
GROUP_SIZE = 16
TOP_K = 8
LANES = 128
BLOCK = 1024
HALF = 8192
GRID = HALF // BLOCK

_MM = functools.partial(jax.lax.dot_general,
                        dimension_numbers=(((1,), (0,)), ((), ())),
                        preferred_element_type=jnp.float32)


def _half_sum_matrix():
    l = np.arange(LANES)
    return jnp.asarray((l[:, None] // 64 == l[None, :] // 64),
                       dtype=np.float32)


def _proj_matrices():
    p1 = np.zeros((LANES, 16), np.float32)
    p2 = np.zeros((LANES, 16), np.float32)
    for c in range(16):
        h, j = divmod(c, 8)
        g, rank = divmod(j, 2)
        (p1 if rank == 0 else p2)[64 * h + 16 * g, c] = 1.0
    return jnp.asarray(p1), jnp.asarray(p2)


def _router_body(xa_ref, xb_ref, j2_ref, p1_ref, p2_ref,
                 rw_ref, tw_ref, ids_ref, hist_ref, acc_ref):
    x = jnp.concatenate([xa_ref[...], xb_ref[...]], axis=1)
    e = jnp.exp(x)

    lane = jax.lax.broadcasted_iota(jnp.int32, (1, LANES), 1)
    lane64 = lane & 63
    lanekey = 63 - lane64
    key = (jax.lax.bitcast_convert_type(e, jnp.int32) & ~63) | lanekey
    keyf = jax.lax.bitcast_convert_type(key, jnp.float32)

    def seg_max(k):
        for s in (1, 2, 4, 8):
            idx = jax.lax.broadcasted_iota(jnp.int32, k.shape, 1) ^ s
            k = jnp.maximum(k, jnp.take_along_axis(k, idx, axis=1))
        return k

    k1 = jax.lax.bitcast_convert_type(seg_max(keyf), jnp.int32)
    i1 = 63 - (k1 & 63)
    m1 = jax.lax.bitcast_convert_type(k1 & ~63, jnp.float32)
    is1 = lane64 == i1
    k2 = jax.lax.bitcast_convert_type(
        seg_max(jnp.where(is1, 0.0, keyf)), jnp.int32)
    i2 = 63 - (k2 & 63)
    m2 = jax.lax.bitcast_convert_type(k2 & ~63, jnp.float32)
    is2 = lane64 == i2
    sel = jnp.where(is1 | is2, 1.0, 0.0).astype(jnp.float32)

    j2 = j2_ref[...]
    rowsum = _MM(e, j2)
    rw = e / rowsum
    rw_ref[0] = rw[:, :64]
    rw_ref[1] = rw[:, 64:]

    den = _MM(sel * e, j2)
    rden = 1.0 / den
    p1 = p1_ref[...]
    p2 = p2_ref[...]
    tw16 = _MM(m1 * rden, p1) + _MM(m2 * rden, p2)
    tw_ref[0] = tw16[:, :8]
    tw_ref[1] = tw16[:, 8:]
    idsf = _MM(i1.astype(jnp.float32), p1) + _MM(i2.astype(jnp.float32), p2)
    ids16 = idsf.astype(jnp.int32)
    ids_ref[0] = ids16[:, :8]
    ids_ref[1] = ids16[:, 8:]

    @pl.when(pl.program_id(0) == 0)
    def _():
        acc_ref[...] = jnp.zeros_like(acc_ref)

    acc_ref[...] += jnp.sum(sel, axis=0, keepdims=True)

    @pl.when(pl.program_id(0) == pl.num_programs(0) - 1)
    def _():
        acc = acc_ref[...]
        hist_ref[...] = acc[:, :64] + acc[:, 64:]


NC = 2
NS = 16
NW = NC * NS


def _sc_router(x_flat, seq):
    from jax.experimental.pallas import tpu_sc as plsc
    tpw = seq // NW

    mesh = plsc.VectorSubcoreMesh(core_axis_name="c", subcore_axis_name="s")

    @functools.partial(
        pl.kernel,
        mesh=mesh,
        out_type=[
            jax.ShapeDtypeStruct((seq * 64,), jnp.float32),
            jax.ShapeDtypeStruct((seq * 8,), jnp.float32),
            jax.ShapeDtypeStruct((seq * 8,), jnp.int32),
            jax.ShapeDtypeStruct((NW * 64,), jnp.float32),
        ],
        compiler_params=pltpu.CompilerParams(needs_layout_passes=False),
        scratch_types=[
            pltpu.VMEM((tpw * 64,), jnp.float32),
            pltpu.VMEM((tpw * 64,), jnp.float32),
            pltpu.VMEM((tpw * 8,), jnp.float32),
            pltpu.VMEM((tpw * 8,), jnp.int32),
            pltpu.VMEM((64,), jnp.float32),
        ],
    )
    def sc_k(x_hbm, rw_hbm, tw_hbm, ids_hbm, hist_hbm,
             x_v, rw_v, tw_v, ids_v, hist_v):
        wid = jax.lax.axis_index("s") * NC + jax.lax.axis_index("c")
        pltpu.sync_copy(x_hbm.at[pl.ds(wid * tpw * 64, tpw * 64)], x_v)
        zero16 = jnp.zeros((16,), jnp.float32)
        for i in range(4):
            hist_v[pl.ds(i * 16, 16)] = zero16
        iota = jax.lax.iota(jnp.int32, 16)
        lt2 = iota < 2
        ones16 = jnp.ones((16,), jnp.float32)

        mask63 = jnp.full((16,), ~63, jnp.int32)

        def body(t, carry):
            base = t * 64
            evs = []
            keys = []
            tot = jnp.float32(0.0)
            for g in range(4):
                v = x_v[pl.ds(base + g * 16, 16)]
                ev = jnp.exp(v)
                evs.append(ev)
                tot = tot + jnp.sum(ev)
                kb = jax.lax.bitcast_convert_type(ev, jnp.int32)
                keys.append(jax.lax.bitcast_convert_type(
                    (kb & mask63) | (63 - iota), jnp.float32))
            r = jnp.ones((16,), jnp.float32) / tot
            svs = []
            sis = []
            sels = []
            den = jnp.float32(0.0)
            for g in range(4):
                rw_v[pl.ds(base + g * 16, 16)] = evs[g] * r
                kv, si = plsc.sort_key_val(keys[g], iota + g * 16,
                                           descending=True)
                sve = jax.lax.bitcast_convert_type(
                    jax.lax.bitcast_convert_type(kv, jnp.int32) & mask63,
                    jnp.float32)
                thr = jnp.sum(jnp.where(iota == 1, kv, zero16))
                sels.append(keys[g] >= thr)
                svs.append(sve)
                sis.append(si)
                den = den + jnp.sum(jnp.where(lt2, sve, zero16))
            rd = jnp.ones((16,), jnp.float32) / den
            t8 = t * 8
            hs = []
            for g in range(4):
                dest = t8 + 2 * g + iota
                plsc.store_scatter(tw_v, [dest], svs[g] * rd, mask=lt2)
                plsc.store_scatter(ids_v, [dest], sis[g], mask=lt2)
                hs.append(carry[g]
                          + jnp.where(sels[g], 1.0, 0.0).astype(jnp.float32))
            return tuple(hs)

        hists = plsc.parallel_loop(0, tpw, unroll=2,
                                   carry=(zero16, zero16, zero16, zero16))(
                                       body)
        for g in range(4):
            hist_v[pl.ds(g * 16, 16)] = hists[g]
        pltpu.sync_copy(rw_v, rw_hbm.at[pl.ds(wid * tpw * 64, tpw * 64)])
        pltpu.sync_copy(tw_v, tw_hbm.at[pl.ds(wid * tpw * 8, tpw * 8)])
        pltpu.sync_copy(ids_v, ids_hbm.at[pl.ds(wid * tpw * 8, tpw * 8)])
        pltpu.sync_copy(hist_v, hist_hbm.at[pl.ds(wid * 64, 64)])

    return sc_k(x_flat)


def _sc_router_partial(x_flat, sc_tokens):
    return _sc_router(x_flat, sc_tokens)


def _hist_reduce_body(parts_ref, out_ref):
    out_ref[...] = jnp.sum(parts_ref[...], axis=0, keepdims=True)


def _hist_reduce(parts):
    return pl.pallas_call(
        _hist_reduce_body,
        out_shape=jax.ShapeDtypeStruct((1, N_EXPERTS), jnp.float32),
    )(parts)


def _kernel_sc_all(logits):
    seq = logits.shape[0]
    x_flat = logits.reshape(seq * 64)
    rw, tw, ids, hist_parts = _sc_router(x_flat, seq)
    hist = _hist_reduce(hist_parts.reshape(NW, 64))
    return (logits,
            rw.reshape(seq, N_EXPERTS),
            tw.reshape(seq, TOP_K),
            ids.reshape(seq, TOP_K),
            hist.reshape(N_EXPERTS))


def _tc_router(logits, start, count, block):
    half = count // 2
    gridn = half // block
    s_blk = start // block
    rw, tw, ids, hist = pl.pallas_call(
        _router_body,
        grid=(gridn,),
        in_specs=[
            pl.BlockSpec((block, N_EXPERTS), lambda i: (s_blk + i, 0)),
            pl.BlockSpec((block, N_EXPERTS),
                         lambda i: (s_blk + gridn + i, 0)),
            pl.BlockSpec((LANES, LANES), lambda i: (0, 0)),
            pl.BlockSpec((LANES, 16), lambda i: (0, 0)),
            pl.BlockSpec((LANES, 16), lambda i: (0, 0)),
        ],
        out_specs=[
            pl.BlockSpec((2, block, N_EXPERTS), lambda i: (0, i, 0)),
            pl.BlockSpec((2, block, TOP_K), lambda i: (0, i, 0)),
            pl.BlockSpec((2, block, TOP_K), lambda i: (0, i, 0)),
            pl.BlockSpec((1, N_EXPERTS), lambda i: (0, 0)),
        ],
        out_shape=[
            jax.ShapeDtypeStruct((2, half, N_EXPERTS), jnp.float32),
            jax.ShapeDtypeStruct((2, half, TOP_K), jnp.float32),
            jax.ShapeDtypeStruct((2, half, TOP_K), jnp.int32),
            jax.ShapeDtypeStruct((1, N_EXPERTS), jnp.float32),
        ],
        scratch_shapes=[pltpu.VMEM((1, LANES), jnp.float32)],
    )(logits, logits, _half_sum_matrix(), *_proj_matrices())
    return (rw.reshape(count, N_EXPERTS),
            tw.reshape(count, TOP_K),
            ids.reshape(count, TOP_K),
            hist)


SC_TOKENS = 2048


def _hist_merge_body(parts_ref, tc_ref, out_ref):
    out_ref[...] = (jnp.sum(parts_ref[...], axis=0, keepdims=True)
                    + tc_ref[...])


def kernel(logits):
    seq = logits.shape[0]
    x_flat = logits.reshape(seq * 64)
    rw_s, tw_s, ids_s, hist_parts = _sc_router(x_flat, SC_TOKENS)
    rw_t, tw_t, ids_t, hist_t = _tc_router(
        logits, SC_TOKENS, seq - SC_TOKENS, 512)
    hist = pl.pallas_call(
        _hist_merge_body,
        out_shape=jax.ShapeDtypeStruct((1, N_EXPERTS), jnp.float32),
    )(hist_parts.reshape(NW, 64), hist_t)
    return (logits,
            jnp.concatenate([rw_s.reshape(SC_TOKENS, N_EXPERTS), rw_t]),
            jnp.concatenate([tw_s.reshape(SC_TOKENS, TOP_K), tw_t]),
            jnp.concatenate([ids_s.reshape(SC_TOKENS, TOP_K), ids_t]),
            hist.reshape(N_EXPERTS))


# speedup vs baseline: 1.0469x; 1.0469x over previous
import functools

import jax
import jax.numpy as jnp
import numpy as np
from jax.experimental import pallas as pl
from jax.experimental.pallas import tpu as pltpu

N_EXPERTS = 64
N_GROUPS = 4
GROUP_SIZE = 16
TOP_K = 8
LANES = 128
BLOCK = 1024
HALF = 8192
GRID = HALF // BLOCK

_MM = functools.partial(jax.lax.dot_general,
                        dimension_numbers=(((1,), (0,)), ((), ())),
                        preferred_element_type=jnp.float32)


def _half_sum_matrix():
    l = np.arange(LANES)
    return jnp.asarray((l[:, None] // 64 == l[None, :] // 64),
                       dtype=np.float32)


def _proj_matrices():
    p1 = np.zeros((LANES, 16), np.float32)
    p2 = np.zeros((LANES, 16), np.float32)
    for c in range(16):
        h, j = divmod(c, 8)
        g, rank = divmod(j, 2)
        (p1 if rank == 0 else p2)[64 * h + 16 * g, c] = 1.0
    return jnp.asarray(p1), jnp.asarray(p2)


def _router_body(xa_ref, xb_ref, j2_ref, p1_ref, p2_ref,
                 rw_ref, tw_ref, ids_ref, hist_ref, acc_ref):
    x = jnp.concatenate([xa_ref[...], xb_ref[...]], axis=1)
    e = jnp.exp(x)

    lane = jax.lax.broadcasted_iota(jnp.int32, (1, LANES), 1)
    lane64 = lane & 63
    lanekey = 63 - lane64
    key = (jax.lax.bitcast_convert_type(e, jnp.int32) & ~63) | lanekey
    keyf = jax.lax.bitcast_convert_type(key, jnp.float32)

    def seg_max(k):
        for s in (1, 2, 4, 8):
            idx = jax.lax.broadcasted_iota(jnp.int32, k.shape, 1) ^ s
            k = jnp.maximum(k, jnp.take_along_axis(k, idx, axis=1))
        return k

    k1 = jax.lax.bitcast_convert_type(seg_max(keyf), jnp.int32)
    i1 = 63 - (k1 & 63)
    m1 = jax.lax.bitcast_convert_type(k1 & ~63, jnp.float32)
    is1 = lane64 == i1
    k2 = jax.lax.bitcast_convert_type(
        seg_max(jnp.where(is1, 0.0, keyf)), jnp.int32)
    i2 = 63 - (k2 & 63)
    m2 = jax.lax.bitcast_convert_type(k2 & ~63, jnp.float32)
    is2 = lane64 == i2
    sel = jnp.where(is1 | is2, 1.0, 0.0).astype(jnp.float32)

    j2 = j2_ref[...]
    rowsum = _MM(e, j2)
    rw = e / rowsum
    rw_ref[0] = rw[:, :64]
    rw_ref[1] = rw[:, 64:]

    den = _MM(sel * e, j2)
    rden = 1.0 / den
    p1 = p1_ref[...]
    p2 = p2_ref[...]
    tw16 = _MM(m1 * rden, p1) + _MM(m2 * rden, p2)
    tw_ref[0] = tw16[:, :8]
    tw_ref[1] = tw16[:, 8:]
    idsf = _MM(i1.astype(jnp.float32), p1) + _MM(i2.astype(jnp.float32), p2)
    ids16 = idsf.astype(jnp.int32)
    ids_ref[0] = ids16[:, :8]
    ids_ref[1] = ids16[:, 8:]

    @pl.when(pl.program_id(0) == 0)
    def _():
        acc_ref[...] = jnp.zeros_like(acc_ref)

    acc_ref[...] += jnp.sum(sel, axis=0, keepdims=True)

    @pl.when(pl.program_id(0) == pl.num_programs(0) - 1)
    def _():
        acc = acc_ref[...]
        hist_ref[...] = acc[:, :64] + acc[:, 64:]


NC = 2
NS = 16
NW = NC * NS


def _sc_router(x_flat, seq):
    from jax.experimental.pallas import tpu_sc as plsc
    tpw = seq // NW

    mesh = plsc.VectorSubcoreMesh(core_axis_name="c", subcore_axis_name="s")

    @functools.partial(
        pl.kernel,
        mesh=mesh,
        out_type=[
            jax.ShapeDtypeStruct((seq * 64,), jnp.float32),
            jax.ShapeDtypeStruct((seq * 8,), jnp.float32),
            jax.ShapeDtypeStruct((seq * 8,), jnp.int32),
            jax.ShapeDtypeStruct((NW * 64,), jnp.float32),
        ],
        compiler_params=pltpu.CompilerParams(needs_layout_passes=False),
        scratch_types=[
            pltpu.VMEM((tpw * 64,), jnp.float32),
            pltpu.VMEM((tpw * 64,), jnp.float32),
            pltpu.VMEM((tpw * 8,), jnp.float32),
            pltpu.VMEM((tpw * 8,), jnp.int32),
            pltpu.VMEM((64,), jnp.float32),
        ],
    )
    def sc_k(x_hbm, rw_hbm, tw_hbm, ids_hbm, hist_hbm,
             x_v, rw_v, tw_v, ids_v, hist_v):
        wid = jax.lax.axis_index("s") * NC + jax.lax.axis_index("c")
        pltpu.sync_copy(x_hbm.at[pl.ds(wid * tpw * 64, tpw * 64)], x_v)
        zero16 = jnp.zeros((16,), jnp.float32)
        for i in range(4):
            hist_v[pl.ds(i * 16, 16)] = zero16
        iota = jax.lax.iota(jnp.int32, 16)
        lt2 = iota < 2
        ones16 = jnp.ones((16,), jnp.float32)

        mask63 = jnp.full((16,), ~63, jnp.int32)

        def body(t, carry):
            base = t * 64
            evs = []
            keys = []
            tot = jnp.float32(0.0)
            for g in range(4):
                v = x_v[pl.ds(base + g * 16, 16)]
                ev = jnp.exp(v)
                evs.append(ev)
                tot = tot + jnp.sum(ev)
                kb = jax.lax.bitcast_convert_type(ev, jnp.int32)
                keys.append(jax.lax.bitcast_convert_type(
                    (kb & mask63) | (63 - iota), jnp.float32))
            r = jnp.ones((16,), jnp.float32) / tot
            svs = []
            sis = []
            sels = []
            den = jnp.float32(0.0)
            for g in range(4):
                rw_v[pl.ds(base + g * 16, 16)] = evs[g] * r
                kv, si = plsc.sort_key_val(keys[g], iota + g * 16,
                                           descending=True)
                sve = jax.lax.bitcast_convert_type(
                    jax.lax.bitcast_convert_type(kv, jnp.int32) & mask63,
                    jnp.float32)
                thr = jnp.sum(jnp.where(iota == 1, kv, zero16))
                sels.append(keys[g] >= thr)
                svs.append(sve)
                sis.append(si)
                den = den + jnp.sum(jnp.where(lt2, sve, zero16))
            rd = jnp.ones((16,), jnp.float32) / den
            t8 = t * 8
            hs = []
            for g in range(4):
                dest = t8 + 2 * g + iota
                plsc.store_scatter(tw_v, [dest], svs[g] * rd, mask=lt2)
                plsc.store_scatter(ids_v, [dest], sis[g], mask=lt2)
                hs.append(carry[g]
                          + jnp.where(sels[g], 1.0, 0.0).astype(jnp.float32))
            return tuple(hs)

        hists = plsc.parallel_loop(0, tpw, unroll=2,
                                   carry=(zero16, zero16, zero16, zero16))(
                                       body)
        for g in range(4):
            hist_v[pl.ds(g * 16, 16)] = hists[g]
        pltpu.sync_copy(rw_v, rw_hbm.at[pl.ds(wid * tpw * 64, tpw * 64)])
        pltpu.sync_copy(tw_v, tw_hbm.at[pl.ds(wid * tpw * 8, tpw * 8)])
        pltpu.sync_copy(ids_v, ids_hbm.at[pl.ds(wid * tpw * 8, tpw * 8)])
        pltpu.sync_copy(hist_v, hist_hbm.at[pl.ds(wid * 64, 64)])

    return sc_k(x_flat)


def _sc_router_partial(x_flat, sc_tokens):
    return _sc_router(x_flat, sc_tokens)


def _hist_reduce_body(parts_ref, out_ref):
    out_ref[...] = jnp.sum(parts_ref[...], axis=0, keepdims=True)


def _hist_reduce(parts):
    return pl.pallas_call(
        _hist_reduce_body,
        out_shape=jax.ShapeDtypeStruct((1, N_EXPERTS), jnp.float32),
    )(parts)


def _kernel_sc_all(logits):
    seq = logits.shape[0]
    x_flat = logits.reshape(seq * 64)
    rw, tw, ids, hist_parts = _sc_router(x_flat, seq)
    hist = _hist_reduce(hist_parts.reshape(NW, 64))
    return (logits,
            rw.reshape(seq, N_EXPERTS),
            tw.reshape(seq, TOP_K),
            ids.reshape(seq, TOP_K),
            hist.reshape(N_EXPERTS))


def _tc_router(logits, start, count, block):
    half = count // 2
    gridn = half // block
    s_blk = start // block
    rw, tw, ids, hist = pl.pallas_call(
        _router_body,
        grid=(gridn,),
        in_specs=[
            pl.BlockSpec((block, N_EXPERTS), lambda i: (s_blk + i, 0)),
            pl.BlockSpec((block, N_EXPERTS),
                         lambda i: (s_blk + gridn + i, 0)),
            pl.BlockSpec((LANES, LANES), lambda i: (0, 0)),
            pl.BlockSpec((LANES, 16), lambda i: (0, 0)),
            pl.BlockSpec((LANES, 16), lambda i: (0, 0)),
        ],
        out_specs=[
            pl.BlockSpec((2, block, N_EXPERTS), lambda i: (0, i, 0)),
            pl.BlockSpec((2, block, TOP_K), lambda i: (0, i, 0)),
            pl.BlockSpec((2, block, TOP_K), lambda i: (0, i, 0)),
            pl.BlockSpec((1, N_EXPERTS), lambda i: (0, 0)),
        ],
        out_shape=[
            jax.ShapeDtypeStruct((2, half, N_EXPERTS), jnp.float32),
            jax.ShapeDtypeStruct((2, half, TOP_K), jnp.float32),
            jax.ShapeDtypeStruct((2, half, TOP_K), jnp.int32),
            jax.ShapeDtypeStruct((1, N_EXPERTS), jnp.float32),
        ],
        scratch_shapes=[pltpu.VMEM((1, LANES), jnp.float32)],
    )(logits, logits, _half_sum_matrix(), *_proj_matrices())
    return (rw.reshape(count, N_EXPERTS),
            tw.reshape(count, TOP_K),
            ids.reshape(count, TOP_K),
            hist)


SC_TOKENS = 8192


def _hist_merge_body(parts_ref, tc_ref, out_ref):
    out_ref[...] = (jnp.sum(parts_ref[...], axis=0, keepdims=True)
                    + tc_ref[...])


def kernel(logits):
    seq = logits.shape[0]
    x_flat = logits.reshape(seq * 64)
    rw_s, tw_s, ids_s, hist_parts = _sc_router(x_flat, SC_TOKENS)
    rw_t, tw_t, ids_t, hist_t = _tc_router(
        logits, SC_TOKENS, seq - SC_TOKENS, 512)
    hist = pl.pallas_call(
        _hist_merge_body,
        out_shape=jax.ShapeDtypeStruct((1, N_EXPERTS), jnp.float32),
    )(hist_parts.reshape(NW, 64), hist_t)
    return (logits,
            jnp.concatenate([rw_s.reshape(SC_TOKENS, N_EXPERTS), rw_t]),
            jnp.concatenate([tw_s.reshape(SC_TOKENS, TOP_K), tw_t]),
            jnp.concatenate([ids_s.reshape(SC_TOKENS, TOP_K), ids_t]),
            hist.reshape(N_EXPERTS))
